# baseline (speedup 1.0000x reference)
import jax
import jax.numpy as jnp
from jax import lax
from jax.experimental import pallas as pl
from jax.experimental.pallas import tpu as pltpu
from jax.experimental.pallas import tpu_sc as plsc

NUM_BINS = 256
XL, XH, YL, YH = 0.0, 1024.0, 0.0, 1024.0
NUM_NETS = 16384
NUM_NODES = 20000
NUM_MOVABLE = 16384
NUM_PINS = 65536
BIN = (XH - XL) / NUM_BINS
BIN_AREA = BIN * BIN
CAP_H = 0.1
MAX_RATE = 2.0
MIN_RATE = 0.5

TN = 2048
NT = NUM_NETS // TN

W = 32
CHUNK = NUM_PINS // W
NV = CHUNK // 16
DUMMY = NUM_NETS
OUT_PAD = 16392
FNP_PAD = NUM_PINS + 16
MARKER_N = 66048
NPS_PAD = 18432
NPS_FILL = 66040
BIG = 3e38


_GDN = lax.GatherDimensionNumbers(
    offset_dims=(), collapsed_slice_dims=(0,), start_index_map=(0,))


def _vtake(v, idx):
    return lax.gather(v, idx[:, None], _GDN, (1,),
                      mode=lax.GatherScatterMode.PROMISE_IN_BOUNDS)


def _sc_bbox_body(pinx_h, piny_h, fnp3_h, fnpf_h, nps_h,
                  bxm_h, bxM_h, bym_h, byM_h,
                  fnp2, gx, gy, seg_a, mbuf, zbuf, nbA, nbB, hbA, hbB,
                  idsA, xmA, xMA, ymA, yMA, idsB, xmB, xMB, ymB, yMB,
                  widx, wx, wy, pidx, pxm_b, pxM_b, pym_b, pyM_b, tmp16,
                  marker_sp, hist_sp, semx, semy, semw, semp, semq):
    nc = 2
    sid = lax.axis_index("s")
    wid = sid * nc + lax.axis_index("c")
    base = wid * CHUNK
    lane = lax.iota(jnp.int32, 16)
    i0 = jnp.zeros((16,), jnp.int32)
    bufs = ((idsA, xmA, xMA, ymA, yMA), (idsB, xmB, xMB, ymB, yMB))
    nbufs = (nbA, nbB)
    hbufs = (hbA, hbB)

    pltpu.sync_copy(fnp3_h.at[wid], fnp2)
    coord_cps = []
    for j in range(16):
        coord_cps.append(pltpu.async_copy(
            pinx_h.at[fnp2.at[j]], gx.at[pl.ds(j * 128, 128)], semx))
        coord_cps.append(pltpu.async_copy(
            piny_h.at[fnp2.at[j]], gy.at[pl.ds(j * 128, 128)], semy))

    def zb(k, _):
        zbuf[pl.ds(k * 16, 16)] = i0
        return 0
    lax.fori_loop(0, 258, zb, 0)
    pltpu.sync_copy(zbuf, marker_sp.at[pl.ds(pl.multiple_of(sid * 4128, 8),
                                             4128)])

    @pl.when(sid == 0)
    def _():
        pltpu.sync_copy(zbuf.at[pl.ds(0, 48)], hist_sp)

    plsc.subcore_barrier()

    ones = i0 + 1
    def obf(k, _):
        zbuf[pl.ds(k * 16, 16)] = ones
        return 0
    lax.fori_loop(0, 8, obf, 0)
    mk_cps = []
    for b in range(9):
        nb = nbufs[b % 2]
        hb = hbufs[b % 2]
        if b >= 2:
            for h in mk_cps[b - 2]:
                h.wait()
        pltpu.sync_copy(
            nps_h.at[pl.ds(pl.multiple_of(sid * 1152 + b * 128, 8), 128)],
            nb)
        def hix(k, _):
            v = nb[pl.ds(k * 16, 16)]
            hb[pl.ds(k * 16, 16)] = lax.shift_right_logical(v, 11)
            return 0
        lax.fori_loop(0, 8, hix, 0)
        h1 = pltpu.async_copy(
            zbuf.at[pl.ds(0, 128)], marker_sp.at[nb], semw, add=True)
        h2 = pltpu.async_copy(
            zbuf.at[pl.ds(0, 128)], hist_sp.at[hb], semw, add=True)
        mk_cps.append((h1, h2))
    for hs in mk_cps[7:]:
        for h in hs:
            h.wait()
    plsc.subcore_barrier()

    pltpu.sync_copy(
        marker_sp.at[pl.ds(pl.multiple_of(base, 8), CHUNK + 16)], mbuf)
    pltpu.sync_copy(hist_sp, zbuf.at[pl.ds(0, 48)])
    h0 = zbuf[pl.ds(0, 16)]
    h1 = zbuf[pl.ds(16, 16)]
    wsp = i0 + wid
    hsum = jnp.where(lane < wsp, h0, 0) + jnp.where(lane + 16 < wsp, h1, 0)
    for s in (1, 2, 4, 8):
        hsum = hsum + _vtake(hsum, jnp.bitwise_xor(lane, s))
    cnt_base = hsum[0]

    def pv(k, c):
        off = k * 16
        v = mbuf[pl.ds(off, 16)]
        for s in (1, 2, 4, 8):
            sh = _vtake(v, jnp.maximum(lane - s, 0))
            v = jnp.where(lane >= s, v + sh, v)
        v = v + c
        seg_a[pl.ds(off, 16)] = v
        return i0 + v[15]
    lax.fori_loop(0, NV + 1, pv, i0 + (cnt_base - 1))

    m0vec = mbuf[pl.ds(0, 16)]
    crosses = (m0vec[0] == 0).astype(jnp.int32)
    s0vec = seg_a[pl.ds(0, 16)]
    gp = s0vec[0]

    tmp16[pl.ds(0, 16)] = (i0 + gp) + jnp.minimum(lane, 1)
    pltpu.async_copy(nps_h.at[tmp16], widx, semp).wait()
    wv = widx[...]
    s_p = wv[0]
    e_p = wv[1]
    a0 = jnp.bitwise_and(s_p, -8)
    nwin = crosses * lax.shift_right_logical(e_p - a0 + 15, 4)

    f16 = lambda v: jnp.full((16,), v, jnp.float32)

    def pwin(j, acc):
        axm, axM, aym, ayM = acc
        wbase = pl.multiple_of(a0 + j * 16, 8)
        pltpu.sync_copy(fnpf_h.at[pl.ds(wbase, 16)], widx)
        pltpu.async_copy(pinx_h.at[widx], wx, semp).wait()
        pltpu.async_copy(piny_h.at[widx], wy, semp).wait()
        pp = wbase + lane
        x = wx[...]
        y = wy[...]
        sel = lambda v, fill: jnp.where(
            pp >= s_p, jnp.where(pp < e_p, v, fill), fill)
        return (jnp.minimum(axm, sel(x, BIG)),
                jnp.maximum(axM, sel(x, -BIG)),
                jnp.minimum(aym, sel(y, BIG)),
                jnp.maximum(ayM, sel(y, -BIG)))

    pxm, pxM, pym, pyM = lax.fori_loop(
        0, nwin, pwin, (f16(BIG), f16(-BIG), f16(BIG), f16(-BIG)))
    for s in (1, 2, 4, 8):
        bf = jnp.bitwise_xor(lane, s)
        pxm = jnp.minimum(pxm, _vtake(pxm, bf))
        pxM = jnp.maximum(pxM, _vtake(pxM, bf))
        pym = jnp.minimum(pym, _vtake(pym, bf))
        pyM = jnp.maximum(pyM, _vtake(pyM, bf))
    inner0 = jnp.where((i0 + crosses) > 0, i0 + gp, DUMMY)
    pidx[pl.ds(0, 16)] = jnp.where(lane == 0, inner0, DUMMY)
    pxm_b[pl.ds(0, 16)] = pxm
    pxM_b[pl.ds(0, 16)] = pxM
    pym_b[pl.ds(0, 16)] = pym
    pyM_b[pl.ds(0, 16)] = pyM
    pcps = [pltpu.async_copy(pxm_b, bxm_h.at[pidx], semq),
            pltpu.async_copy(pxM_b, bxM_h.at[pidx], semq),
            pltpu.async_copy(pym_b, bym_h.at[pidx], semq),
            pltpu.async_copy(pyM_b, byM_h.at[pidx], semq)]

    for cp in coord_cps:
        cp.wait()

    gpv = i0 + gp
    crvi = i0 + crosses
    carry = (i0 - 1, f16(BIG), f16(-BIG), f16(BIG), f16(-BIG))
    out_cps = []
    for blk in range(16):
        ids_b, xm_b, xM_b, ym_b, yM_b = bufs[blk % 2]
        if blk >= 2:
            for h in out_cps[blk - 2]:
                h.wait()

        def mainb(kl, c):
            cgv, cxm, cxM, cym, cyM = c
            off = blk * 128 + kl * 16
            x = gx[pl.ds(off, 16)]
            y = gy[pl.ds(off, 16)]
            g = seg_a[pl.ds(off, 16)]
            gnv = seg_a[pl.ds(off + 16, 16)]
            xm, xM, ym, yM = x, x, y, y
            for s in (1, 2, 4, 8):
                idxs = jnp.maximum(lane - s, 0)
                pen = jnp.maximum(jnp.full((16,), s, jnp.int32) - lane, 0)
                ok = (jnp.abs(_vtake(g, idxs) - g) + pen) == 0
                xm = jnp.where(ok, jnp.minimum(xm, _vtake(xm, idxs)), xm)
                xM = jnp.where(ok, jnp.maximum(xM, _vtake(xM, idxs)), xM)
                ym = jnp.where(ok, jnp.minimum(ym, _vtake(ym, idxs)), ym)
                yM = jnp.where(ok, jnp.maximum(yM, _vtake(yM, idxs)), yM)
            mc = g == cgv
            xm = jnp.where(mc, jnp.minimum(xm, cxm), xm)
            xM = jnp.where(mc, jnp.maximum(xM, cxM), xM)
            ym = jnp.where(mc, jnp.minimum(ym, cym), ym)
            yM = jnp.where(mc, jnp.maximum(yM, cyM), yM)
            gnext = jnp.where(lane == 15, i0 + gnv[0],
                              _vtake(g, jnp.minimum(lane + 1, 15)))
            suppress = jnp.where(g == gpv, crvi, i0)
            me_i = jnp.where(gnext != g, 1 - suppress, i0)
            o = kl * 16
            ids_b[pl.ds(o, 16)] = jnp.where(me_i > 0, g, DUMMY)
            xm_b[pl.ds(o, 16)] = xm
            xM_b[pl.ds(o, 16)] = xM
            ym_b[pl.ds(o, 16)] = ym
            yM_b[pl.ds(o, 16)] = yM
            return (i0 + g[15], f16(0.0) + xm[15], f16(0.0) + xM[15],
                    f16(0.0) + ym[15], f16(0.0) + yM[15])

        carry = lax.fori_loop(0, 8, mainb, carry)
        out_cps.append([
            pltpu.async_copy(xm_b, bxm_h.at[ids_b], semw),
            pltpu.async_copy(xM_b, bxM_h.at[ids_b], semw),
            pltpu.async_copy(ym_b, bym_h.at[ids_b], semw),
            pltpu.async_copy(yM_b, byM_h.at[ids_b], semw)])

    for hs in out_cps[14:]:
        for h in hs:
            h.wait()
    for cp in pcps:
        cp.wait()


def _sc_bbox(pin_x, pin_y, flat_netpin, netpin_start):
    fnp3 = flat_netpin.reshape(W, 16, 128)
    fnpf = jnp.concatenate(
        [flat_netpin, jnp.zeros((FNP_PAD - NUM_PINS,), jnp.int32)])
    nps = jnp.concatenate(
        [netpin_start,
         jnp.full((NPS_PAD - NUM_NETS - 1,), NPS_FILL, jnp.int32)])
    mesh = plsc.VectorSubcoreMesh(core_axis_name="c", subcore_axis_name="s")
    f = pl.kernel(
        _sc_bbox_body,
        out_type=[jax.ShapeDtypeStruct((OUT_PAD,), jnp.float32)] * 4,
        mesh=mesh,
        scratch_types=[
            pltpu.VMEM((16, 128), jnp.int32),
            pltpu.VMEM((CHUNK,), jnp.float32),
            pltpu.VMEM((CHUNK,), jnp.float32),
            pltpu.VMEM((CHUNK + 16,), jnp.int32),
            pltpu.VMEM((CHUNK + 16,), jnp.int32),
            pltpu.VMEM((4128,), jnp.int32),
            pltpu.VMEM((128,), jnp.int32),
            pltpu.VMEM((128,), jnp.int32),
            pltpu.VMEM((128,), jnp.int32),
            pltpu.VMEM((128,), jnp.int32),
            pltpu.VMEM((128,), jnp.int32),
            pltpu.VMEM((128,), jnp.float32),
            pltpu.VMEM((128,), jnp.float32),
            pltpu.VMEM((128,), jnp.float32),
            pltpu.VMEM((128,), jnp.float32),
            pltpu.VMEM((128,), jnp.int32),
            pltpu.VMEM((128,), jnp.float32),
            pltpu.VMEM((128,), jnp.float32),
            pltpu.VMEM((128,), jnp.float32),
            pltpu.VMEM((128,), jnp.float32),
            pltpu.VMEM((16,), jnp.int32),
            pltpu.VMEM((16,), jnp.float32),
            pltpu.VMEM((16,), jnp.float32),
            pltpu.VMEM((16,), jnp.int32),
            pltpu.VMEM((16,), jnp.float32),
            pltpu.VMEM((16,), jnp.float32),
            pltpu.VMEM((16,), jnp.float32),
            pltpu.VMEM((16,), jnp.float32),
            pltpu.VMEM((16,), jnp.int32),
            pltpu.VMEM_SHARED((MARKER_N,), jnp.int32),
            pltpu.VMEM_SHARED((48,), jnp.int32),
            pltpu.SemaphoreType.DMA,
            pltpu.SemaphoreType.DMA,
            pltpu.SemaphoreType.DMA,
            pltpu.SemaphoreType.DMA,
            pltpu.SemaphoreType.DMA,
        ],
    )
    return f(pin_x, pin_y, fnp3, fnpf, nps)


def _demand_body(xm_ref, xM_ref, ym_ref, yM_ref, cnt_ref, rt_ref, acc_ref):
    i = pl.program_id(0)

    @pl.when(i == 0)
    def _():
        acc_ref[...] = jnp.zeros_like(acc_ref)

    valid = cnt_ref[0] > 0
    xm = jnp.where(valid, xm_ref[0], 0.0)
    xM = jnp.where(valid, xM_ref[0], 0.0)
    ym = jnp.where(valid, ym_ref[0], 0.0)
    yM = jnp.where(valid, yM_ref[0], 0.0)
    w = xM - xm
    h = yM - ym
    area = w * h
    pos = area > 0
    safe = jnp.where(pos, area, 1.0)
    dh = jnp.where(pos, w / safe, 0.0)
    dv = jnp.where(pos, h / safe, 0.0)
    b_lo = lax.broadcasted_iota(jnp.int32, (NUM_BINS, TN), 0).astype(
        jnp.float32) * BIN
    ox = jnp.clip(jnp.minimum(xM, b_lo + BIN) - jnp.maximum(xm, b_lo), 0.0, BIN)
    oy = jnp.clip(jnp.minimum(yM, b_lo + BIN) - jnp.maximum(ym, b_lo), 0.0, BIN)
    stacked = jnp.concatenate([ox * dh, ox * dv], axis=0)
    acc_ref[...] += lax.dot_general(
        stacked, oy, (((1,), (1,)), ((), ())),
        preferred_element_type=jnp.float32)

    @pl.when(i == NT - 1)
    def _():
        util = acc_ref[...] / (CAP_H * BIN_AREA)
        rt_ref[...] = jnp.clip(
            jnp.maximum(util[:NUM_BINS, :], util[NUM_BINS:, :]),
            MIN_RATE, MAX_RATE)


def _instance_body(rt_ref, nx_ref, ny_ref, sx_ref, sy_ref, out_ref):
    b_lo = lax.broadcasted_iota(jnp.int32, (NUM_BINS, TN), 0).astype(
        jnp.float32) * BIN
    nx = nx_ref[0]
    ny = ny_ref[0]
    nox = jnp.clip(jnp.minimum(nx + sx_ref[0], b_lo + BIN)
                   - jnp.maximum(nx, b_lo), 0.0, BIN)
    noy = jnp.clip(jnp.minimum(ny + sy_ref[0], b_lo + BIN)
                   - jnp.maximum(ny, b_lo), 0.0, BIN)
    t1 = lax.dot_general(rt_ref[...], nox, (((0,), (0,)), ((), ())),
                         preferred_element_type=jnp.float32)
    out_ref[0] = jnp.sum(t1 * noy, axis=0, keepdims=True)


def _dense_pipeline(x_min, x_max, y_min, y_max, counts, nx, ny, sx, sy):
    r2 = lambda a: a.reshape(NT, 1, TN)
    rt = pl.pallas_call(
        _demand_body,
        grid=(NT,),
        in_specs=[pl.BlockSpec((1, 1, TN), lambda i: (i, 0, 0))] * 5,
        out_specs=pl.BlockSpec((NUM_BINS, NUM_BINS), lambda i: (0, 0)),
        out_shape=jax.ShapeDtypeStruct((NUM_BINS, NUM_BINS), jnp.float32),
        scratch_shapes=[pltpu.VMEM((2 * NUM_BINS, NUM_BINS), jnp.float32)],
    )(r2(x_min), r2(x_max), r2(y_min), r2(y_max), r2(counts))
    out = pl.pallas_call(
        _instance_body,
        grid=(NT,),
        in_specs=[pl.BlockSpec((NUM_BINS, NUM_BINS), lambda i: (0, 0))]
        + [pl.BlockSpec((1, 1, TN), lambda i: (i, 0, 0))] * 4,
        out_specs=pl.BlockSpec((1, 1, TN), lambda i: (i, 0, 0)),
        out_shape=jax.ShapeDtypeStruct((NT, 1, TN), jnp.float32),
    )(rt, r2(nx), r2(ny), r2(sx), r2(sy))
    return out.reshape(NUM_MOVABLE)


def kernel(pos, pin_pos, node_size_x, node_size_y, netpin_start, flat_netpin):
    pin_x = pin_pos[:NUM_PINS]
    pin_y = pin_pos[NUM_PINS:]
    counts = netpin_start[1:] - netpin_start[:-1]
    x_min, x_max, y_min, y_max = _sc_bbox(pin_x, pin_y, flat_netpin,
                                          netpin_start)
    x_min = x_min[:NUM_NETS]
    x_max = x_max[:NUM_NETS]
    y_min = y_min[:NUM_NETS]
    y_max = y_max[:NUM_NETS]

    nx = pos[:NUM_MOVABLE]
    ny = pos[NUM_NODES:NUM_NODES + NUM_MOVABLE]
    sx = node_size_x[:NUM_MOVABLE]
    sy = node_size_y[:NUM_MOVABLE]
    return _dense_pipeline(x_min, x_max, y_min, y_max,
                           counts.astype(jnp.float32), nx, ny, sx, sy)


# speedup vs baseline: 12.9556x; 12.9556x over previous
import jax
import jax.numpy as jnp
from jax import lax
from jax.experimental import pallas as pl
from jax.experimental.pallas import tpu as pltpu
from jax.experimental.pallas import tpu_sc as plsc

NUM_BINS = 256
XL, XH, YL, YH = 0.0, 1024.0, 0.0, 1024.0
NUM_NETS = 16384
NUM_NODES = 20000
NUM_MOVABLE = 16384
NUM_PINS = 65536
BIN = (XH - XL) / NUM_BINS
BIN_AREA = BIN * BIN
CAP_H = 0.1
MAX_RATE = 2.0
MIN_RATE = 0.5

TN = 2048
NT = NUM_NETS // TN

W = 32
CHUNK = NUM_PINS // W
NV = CHUNK // 16
DUMMY = NUM_NETS
OUT_PAD = 82440
FNP_PAD = NUM_PINS + 16
MARKER_N = 66048
NPS_PAD = 18432
NPS_FILL = 66040
BIG = 3e38


_GDN = lax.GatherDimensionNumbers(
    offset_dims=(), collapsed_slice_dims=(0,), start_index_map=(0,))


def _vtake(v, idx):
    return lax.gather(v, idx[:, None], _GDN, (1,),
                      mode=lax.GatherScatterMode.PROMISE_IN_BOUNDS)


def _sc_bbox_body(pinx_h, piny_h, fnp3_h, fnpf_h, nps_h,
                  bxm_h, bxM_h, bym_h, byM_h,
                  fnp2, gx, gy, seg_a, mbuf, zbuf, nbA, nbB, hbA, hbB,
                  idsA, xmA, xMA, ymA, yMA, idsB, xmB, xMB, ymB, yMB,
                  widx, wx, wy, pidx, pxm_b, pxM_b, pym_b, pyM_b, tmp16,
                  marker_sp, hist_sp, semx, semy, semw, semp, semq):
    nc = 2
    sid = lax.axis_index("s")
    wid = sid * nc + lax.axis_index("c")
    base = wid * CHUNK
    lane = lax.iota(jnp.int32, 16)
    i0 = jnp.zeros((16,), jnp.int32)
    bufs = ((idsA, xmA, xMA, ymA, yMA), (idsB, xmB, xMB, ymB, yMB))
    nbufs = (nbA, nbB)
    hbufs = (hbA, hbB)

    pltpu.sync_copy(fnp3_h.at[wid], fnp2)
    coord_cps = []
    for j in range(16):
        coord_cps.append(pltpu.async_copy(
            pinx_h.at[fnp2.at[j]], gx.at[pl.ds(j * 128, 128)], semx))
        coord_cps.append(pltpu.async_copy(
            piny_h.at[fnp2.at[j]], gy.at[pl.ds(j * 128, 128)], semy))

    def zb(k, _):
        zbuf[pl.ds(k * 16, 16)] = i0
        return 0
    lax.fori_loop(0, 258, zb, 0)
    pltpu.sync_copy(zbuf, marker_sp.at[pl.ds(pl.multiple_of(sid * 4128, 8),
                                             4128)])

    @pl.when(sid == 0)
    def _():
        pltpu.sync_copy(zbuf.at[pl.ds(0, 48)], hist_sp)

    plsc.subcore_barrier()

    ones = i0 + 1
    def obf(k, _):
        zbuf[pl.ds(k * 16, 16)] = ones
        return 0
    lax.fori_loop(0, 8, obf, 0)
    mk_cps = []
    for b in range(9):
        nb = nbufs[b % 2]
        hb = hbufs[b % 2]
        if b >= 2:
            for h in mk_cps[b - 2]:
                h.wait()
        pltpu.sync_copy(
            nps_h.at[pl.ds(pl.multiple_of(sid * 1152 + b * 128, 8), 128)],
            nb)
        def hix(k, _):
            v = nb[pl.ds(k * 16, 16)]
            hb[pl.ds(k * 16, 16)] = lax.shift_right_logical(v, 11)
            return 0
        lax.fori_loop(0, 8, hix, 0)
        h1 = pltpu.async_copy(
            zbuf.at[pl.ds(0, 128)], marker_sp.at[nb], semw, add=True)
        h2 = pltpu.async_copy(
            zbuf.at[pl.ds(0, 128)], hist_sp.at[hb], semw, add=True)
        mk_cps.append((h1, h2))
    for hs in mk_cps[7:]:
        for h in hs:
            h.wait()
    plsc.subcore_barrier()

    pltpu.sync_copy(
        marker_sp.at[pl.ds(pl.multiple_of(base, 8), CHUNK + 16)], mbuf)
    pltpu.sync_copy(hist_sp, zbuf.at[pl.ds(0, 48)])
    h0 = zbuf[pl.ds(0, 16)]
    h1 = zbuf[pl.ds(16, 16)]
    wsp = i0 + wid
    hsum = jnp.where(lane < wsp, h0, 0) + jnp.where(lane + 16 < wsp, h1, 0)
    for s in (1, 2, 4, 8):
        hsum = hsum + _vtake(hsum, jnp.bitwise_xor(lane, s))
    cnt_base = hsum[0]

    def pv(k, c):
        off = k * 16
        v = mbuf[pl.ds(off, 16)]
        for s in (1, 2, 4, 8):
            sh = _vtake(v, jnp.maximum(lane - s, 0))
            v = jnp.where(lane >= s, v + sh, v)
        v = v + c
        seg_a[pl.ds(off, 16)] = v
        return i0 + v[15]
    lax.fori_loop(0, NV + 1, pv, i0 + (cnt_base - 1))

    m0vec = mbuf[pl.ds(0, 16)]
    crosses = (m0vec[0] == 0).astype(jnp.int32)
    s0vec = seg_a[pl.ds(0, 16)]
    gp = s0vec[0]

    tmp16[pl.ds(0, 16)] = (i0 + gp) + jnp.minimum(lane, 1)
    pltpu.async_copy(nps_h.at[tmp16], widx, semp).wait()
    wv = widx[...]
    s_p = wv[0]
    e_p = wv[1]
    a0 = jnp.bitwise_and(s_p, -8)
    nwin = crosses * lax.shift_right_logical(e_p - a0 + 15, 4)

    f16 = lambda v: jnp.full((16,), v, jnp.float32)

    def pwin(j, acc):
        axm, axM, aym, ayM = acc
        wbase = pl.multiple_of(a0 + j * 16, 8)
        pltpu.sync_copy(fnpf_h.at[pl.ds(wbase, 16)], widx)
        pltpu.async_copy(pinx_h.at[widx], wx, semp).wait()
        pltpu.async_copy(piny_h.at[widx], wy, semp).wait()
        pp = wbase + lane
        x = wx[...]
        y = wy[...]
        sel = lambda v, fill: jnp.where(
            pp >= s_p, jnp.where(pp < e_p, v, fill), fill)
        return (jnp.minimum(axm, sel(x, BIG)),
                jnp.maximum(axM, sel(x, -BIG)),
                jnp.minimum(aym, sel(y, BIG)),
                jnp.maximum(ayM, sel(y, -BIG)))

    pxm, pxM, pym, pyM = lax.fori_loop(
        0, nwin, pwin, (f16(BIG), f16(-BIG), f16(BIG), f16(-BIG)))
    for s in (1, 2, 4, 8):
        bf = jnp.bitwise_xor(lane, s)
        pxm = jnp.minimum(pxm, _vtake(pxm, bf))
        pxM = jnp.maximum(pxM, _vtake(pxM, bf))
        pym = jnp.minimum(pym, _vtake(pym, bf))
        pyM = jnp.maximum(pyM, _vtake(pyM, bf))
    fb = NUM_NETS + NUM_PINS + wid * 16 + lane
    pm = crosses * jnp.maximum(1 - lane, 0)
    pidx[pl.ds(0, 16)] = fb + pm * ((i0 + gp) - fb)
    pxm_b[pl.ds(0, 16)] = pxm
    pxM_b[pl.ds(0, 16)] = pxM
    pym_b[pl.ds(0, 16)] = pym
    pyM_b[pl.ds(0, 16)] = pyM
    pcps = [pltpu.async_copy(pxm_b, bxm_h.at[pidx], semq),
            pltpu.async_copy(pxM_b, bxM_h.at[pidx], semq),
            pltpu.async_copy(pym_b, bym_h.at[pidx], semq),
            pltpu.async_copy(pyM_b, byM_h.at[pidx], semq)]

    for cp in coord_cps:
        cp.wait()

    gpv = i0 + gp
    crvi = i0 + crosses
    carry = (i0 - 1, f16(BIG), f16(-BIG), f16(BIG), f16(-BIG))
    out_cps = []
    for blk in range(16):
        ids_b, xm_b, xM_b, ym_b, yM_b = bufs[blk % 2]
        if blk >= 2:
            for h in out_cps[blk - 2]:
                h.wait()

        def mainb(kl, c):
            cgv, cxm, cxM, cym, cyM = c
            off = blk * 128 + kl * 16
            x = gx[pl.ds(off, 16)]
            y = gy[pl.ds(off, 16)]
            g = seg_a[pl.ds(off, 16)]
            gnv = seg_a[pl.ds(off + 16, 16)]
            xm, xM, ym, yM = x, x, y, y
            for s in (1, 2, 4, 8):
                idxs = jnp.maximum(lane - s, 0)
                pen = jnp.maximum(jnp.full((16,), s, jnp.int32) - lane, 0)
                ok = (jnp.abs(_vtake(g, idxs) - g) + pen) == 0
                xm = jnp.where(ok, jnp.minimum(xm, _vtake(xm, idxs)), xm)
                xM = jnp.where(ok, jnp.maximum(xM, _vtake(xM, idxs)), xM)
                ym = jnp.where(ok, jnp.minimum(ym, _vtake(ym, idxs)), ym)
                yM = jnp.where(ok, jnp.maximum(yM, _vtake(yM, idxs)), yM)
            mc = g == cgv
            xm = jnp.where(mc, jnp.minimum(xm, cxm), xm)
            xM = jnp.where(mc, jnp.maximum(xM, cxM), xM)
            ym = jnp.where(mc, jnp.minimum(ym, cym), ym)
            yM = jnp.where(mc, jnp.maximum(yM, cyM), yM)
            gnext = jnp.where(lane == 15, i0 + gnv[0],
                              _vtake(g, jnp.minimum(lane + 1, 15)))
            suppress = jnp.where(g == gpv, crvi, i0)
            me_i = jnp.where(gnext != g, 1 - suppress, i0)
            o = kl * 16
            fbv = NUM_NETS + base + off + lane
            ids_b[pl.ds(o, 16)] = fbv + me_i * (g - fbv)
            xm_b[pl.ds(o, 16)] = xm
            xM_b[pl.ds(o, 16)] = xM
            ym_b[pl.ds(o, 16)] = ym
            yM_b[pl.ds(o, 16)] = yM
            return (i0 + g[15], f16(0.0) + xm[15], f16(0.0) + xM[15],
                    f16(0.0) + ym[15], f16(0.0) + yM[15])

        carry = lax.fori_loop(0, 8, mainb, carry)
        out_cps.append([
            pltpu.async_copy(xm_b, bxm_h.at[ids_b], semw),
            pltpu.async_copy(xM_b, bxM_h.at[ids_b], semw),
            pltpu.async_copy(ym_b, bym_h.at[ids_b], semw),
            pltpu.async_copy(yM_b, byM_h.at[ids_b], semw)])

    for hs in out_cps[14:]:
        for h in hs:
            h.wait()
    for cp in pcps:
        cp.wait()


def _sc_bbox(pin_x, pin_y, flat_netpin, netpin_start):
    fnp3 = flat_netpin.reshape(W, 16, 128)
    fnpf = jnp.concatenate(
        [flat_netpin, jnp.zeros((FNP_PAD - NUM_PINS,), jnp.int32)])
    nps = jnp.concatenate(
        [netpin_start,
         jnp.full((NPS_PAD - NUM_NETS - 1,), NPS_FILL, jnp.int32)])
    mesh = plsc.VectorSubcoreMesh(core_axis_name="c", subcore_axis_name="s")
    f = pl.kernel(
        _sc_bbox_body,
        out_type=[jax.ShapeDtypeStruct((OUT_PAD,), jnp.float32)] * 4,
        mesh=mesh,
        scratch_types=[
            pltpu.VMEM((16, 128), jnp.int32),
            pltpu.VMEM((CHUNK,), jnp.float32),
            pltpu.VMEM((CHUNK,), jnp.float32),
            pltpu.VMEM((CHUNK + 16,), jnp.int32),
            pltpu.VMEM((CHUNK + 16,), jnp.int32),
            pltpu.VMEM((4128,), jnp.int32),
            pltpu.VMEM((128,), jnp.int32),
            pltpu.VMEM((128,), jnp.int32),
            pltpu.VMEM((128,), jnp.int32),
            pltpu.VMEM((128,), jnp.int32),
            pltpu.VMEM((128,), jnp.int32),
            pltpu.VMEM((128,), jnp.float32),
            pltpu.VMEM((128,), jnp.float32),
            pltpu.VMEM((128,), jnp.float32),
            pltpu.VMEM((128,), jnp.float32),
            pltpu.VMEM((128,), jnp.int32),
            pltpu.VMEM((128,), jnp.float32),
            pltpu.VMEM((128,), jnp.float32),
            pltpu.VMEM((128,), jnp.float32),
            pltpu.VMEM((128,), jnp.float32),
            pltpu.VMEM((16,), jnp.int32),
            pltpu.VMEM((16,), jnp.float32),
            pltpu.VMEM((16,), jnp.float32),
            pltpu.VMEM((16,), jnp.int32),
            pltpu.VMEM((16,), jnp.float32),
            pltpu.VMEM((16,), jnp.float32),
            pltpu.VMEM((16,), jnp.float32),
            pltpu.VMEM((16,), jnp.float32),
            pltpu.VMEM((16,), jnp.int32),
            pltpu.VMEM_SHARED((MARKER_N,), jnp.int32),
            pltpu.VMEM_SHARED((48,), jnp.int32),
            pltpu.SemaphoreType.DMA,
            pltpu.SemaphoreType.DMA,
            pltpu.SemaphoreType.DMA,
            pltpu.SemaphoreType.DMA,
            pltpu.SemaphoreType.DMA,
        ],
    )
    return f(pin_x, pin_y, fnp3, fnpf, nps)


def _demand_body(xm_ref, xM_ref, ym_ref, yM_ref, cnt_ref, rt_ref, acc_ref):
    i = pl.program_id(0)

    @pl.when(i == 0)
    def _():
        acc_ref[...] = jnp.zeros_like(acc_ref)

    valid = cnt_ref[0] > 0
    xm = jnp.where(valid, xm_ref[0], 0.0)
    xM = jnp.where(valid, xM_ref[0], 0.0)
    ym = jnp.where(valid, ym_ref[0], 0.0)
    yM = jnp.where(valid, yM_ref[0], 0.0)
    w = xM - xm
    h = yM - ym
    area = w * h
    pos = area > 0
    safe = jnp.where(pos, area, 1.0)
    dh = jnp.where(pos, w / safe, 0.0)
    dv = jnp.where(pos, h / safe, 0.0)
    b_lo = lax.broadcasted_iota(jnp.int32, (NUM_BINS, TN), 0).astype(
        jnp.float32) * BIN
    ox = jnp.clip(jnp.minimum(xM, b_lo + BIN) - jnp.maximum(xm, b_lo), 0.0, BIN)
    oy = jnp.clip(jnp.minimum(yM, b_lo + BIN) - jnp.maximum(ym, b_lo), 0.0, BIN)
    stacked = jnp.concatenate([ox * dh, ox * dv], axis=0)
    acc_ref[...] += lax.dot_general(
        stacked, oy, (((1,), (1,)), ((), ())),
        preferred_element_type=jnp.float32)

    @pl.when(i == NT - 1)
    def _():
        util = acc_ref[...] / (CAP_H * BIN_AREA)
        rt_ref[...] = jnp.clip(
            jnp.maximum(util[:NUM_BINS, :], util[NUM_BINS:, :]),
            MIN_RATE, MAX_RATE)


def _instance_body(rt_ref, nx_ref, ny_ref, sx_ref, sy_ref, out_ref):
    b_lo = lax.broadcasted_iota(jnp.int32, (NUM_BINS, TN), 0).astype(
        jnp.float32) * BIN
    nx = nx_ref[0]
    ny = ny_ref[0]
    nox = jnp.clip(jnp.minimum(nx + sx_ref[0], b_lo + BIN)
                   - jnp.maximum(nx, b_lo), 0.0, BIN)
    noy = jnp.clip(jnp.minimum(ny + sy_ref[0], b_lo + BIN)
                   - jnp.maximum(ny, b_lo), 0.0, BIN)
    t1 = lax.dot_general(rt_ref[...], nox, (((0,), (0,)), ((), ())),
                         preferred_element_type=jnp.float32)
    out_ref[0] = jnp.sum(t1 * noy, axis=0, keepdims=True)


def _dense_pipeline(x_min, x_max, y_min, y_max, counts, nx, ny, sx, sy):
    r2 = lambda a: a.reshape(NT, 1, TN)
    rt = pl.pallas_call(
        _demand_body,
        grid=(NT,),
        in_specs=[pl.BlockSpec((1, 1, TN), lambda i: (i, 0, 0))] * 5,
        out_specs=pl.BlockSpec((NUM_BINS, NUM_BINS), lambda i: (0, 0)),
        out_shape=jax.ShapeDtypeStruct((NUM_BINS, NUM_BINS), jnp.float32),
        scratch_shapes=[pltpu.VMEM((2 * NUM_BINS, NUM_BINS), jnp.float32)],
    )(r2(x_min), r2(x_max), r2(y_min), r2(y_max), r2(counts))
    out = pl.pallas_call(
        _instance_body,
        grid=(NT,),
        in_specs=[pl.BlockSpec((NUM_BINS, NUM_BINS), lambda i: (0, 0))]
        + [pl.BlockSpec((1, 1, TN), lambda i: (i, 0, 0))] * 4,
        out_specs=pl.BlockSpec((1, 1, TN), lambda i: (i, 0, 0)),
        out_shape=jax.ShapeDtypeStruct((NT, 1, TN), jnp.float32),
    )(rt, r2(nx), r2(ny), r2(sx), r2(sy))
    return out.reshape(NUM_MOVABLE)


def kernel(pos, pin_pos, node_size_x, node_size_y, netpin_start, flat_netpin):
    pin_x = pin_pos[:NUM_PINS]
    pin_y = pin_pos[NUM_PINS:]
    counts = netpin_start[1:] - netpin_start[:-1]
    x_min, x_max, y_min, y_max = _sc_bbox(pin_x, pin_y, flat_netpin,
                                          netpin_start)
    x_min = x_min[:NUM_NETS]
    x_max = x_max[:NUM_NETS]
    y_min = y_min[:NUM_NETS]
    y_max = y_max[:NUM_NETS]

    nx = pos[:NUM_MOVABLE]
    ny = pos[NUM_NODES:NUM_NODES + NUM_MOVABLE]
    sx = node_size_x[:NUM_MOVABLE]
    sy = node_size_y[:NUM_MOVABLE]
    return _dense_pipeline(x_min, x_max, y_min, y_max,
                           counts.astype(jnp.float32), nx, ny, sx, sy)


# speedup vs baseline: 95.2967x; 7.3556x over previous
import jax
import jax.numpy as jnp
from jax import lax
from jax.experimental import pallas as pl
from jax.experimental.pallas import tpu as pltpu
from jax.experimental.pallas import tpu_sc as plsc

NUM_BINS = 256
XL, XH, YL, YH = 0.0, 1024.0, 0.0, 1024.0
NUM_NETS = 16384
NUM_NODES = 20000
NUM_MOVABLE = 16384
NUM_PINS = 65536
BIN = (XH - XL) / NUM_BINS
BIN_AREA = BIN * BIN
CAP_H = 0.1
MAX_RATE = 2.0
MIN_RATE = 0.5

TN = 2048
NT = NUM_NETS // TN

W = 32
CHUNK = NUM_PINS // W
NV = CHUNK // 16
NREC = NUM_PINS + 64 * W
NT_D = NREC // TN
FNP_PAD = NUM_PINS + 16
MARKER_N = 66048
NPS_PAD = 18432
NPS_FILL = 66040
BIG = 3e38


_GDN = lax.GatherDimensionNumbers(
    offset_dims=(), collapsed_slice_dims=(0,), start_index_map=(0,))


def _vtake(v, idx):
    return lax.gather(v, idx[:, None], _GDN, (1,),
                      mode=lax.GatherScatterMode.PROMISE_IN_BOUNDS)


def _sc_bbox_body(pinx_h, piny_h, fnp3_h, fnpf_h, nps_h,
                  bxm_h, bxM_h, bym_h, byM_h,
                  fnp2, gx, gy, seg_a, mbuf, zbuf, nbA, nbB, hbA, hbB,
                  xmA, xMA, ymA, yMA, xmB, xMB, ymB, yMB,
                  widx, wx, wy, sxm_b, sxM_b, sym_b, syM_b, tmp16,
                  marker_sp, hist_sp, semx, semy, semw, semp, semq):
    nc = 2
    sid = lax.axis_index("s")
    wid = sid * nc + lax.axis_index("c")
    base = wid * CHUNK
    lane = lax.iota(jnp.int32, 16)
    i0 = jnp.zeros((16,), jnp.int32)
    bufs = ((xmA, xMA, ymA, yMA), (xmB, xMB, ymB, yMB))
    nbufs = (nbA, nbB)
    hbufs = (hbA, hbB)

    pltpu.sync_copy(fnp3_h.at[wid], fnp2)
    coord_cps = []
    for j in range(16):
        coord_cps.append(pltpu.async_copy(
            pinx_h.at[fnp2.at[j]], gx.at[pl.ds(j * 128, 128)], semx))
        coord_cps.append(pltpu.async_copy(
            piny_h.at[fnp2.at[j]], gy.at[pl.ds(j * 128, 128)], semy))

    def zb(k, _):
        zbuf[pl.ds(k * 16, 16)] = i0
        return 0
    lax.fori_loop(0, 258, zb, 0)
    pltpu.sync_copy(zbuf, marker_sp.at[pl.ds(pl.multiple_of(sid * 4128, 8),
                                             4128)])

    @pl.when(sid == 0)
    def _():
        pltpu.sync_copy(zbuf.at[pl.ds(0, 48)], hist_sp)

    plsc.subcore_barrier()

    ones = i0 + 1
    def obf(k, _):
        zbuf[pl.ds(k * 16, 16)] = ones
        return 0
    lax.fori_loop(0, 8, obf, 0)
    mk_cps = []
    for b in range(9):
        nb = nbufs[b % 2]
        hb = hbufs[b % 2]
        if b >= 2:
            for h in mk_cps[b - 2]:
                h.wait()
        pltpu.sync_copy(
            nps_h.at[pl.ds(pl.multiple_of(sid * 1152 + b * 128, 8), 128)],
            nb)
        def hix(k, _):
            v = nb[pl.ds(k * 16, 16)]
            hb[pl.ds(k * 16, 16)] = lax.shift_right_logical(v, 11)
            return 0
        lax.fori_loop(0, 8, hix, 0)
        h1 = pltpu.async_copy(
            zbuf.at[pl.ds(0, 128)], marker_sp.at[nb], semw, add=True)
        h2 = pltpu.async_copy(
            zbuf.at[pl.ds(0, 128)], hist_sp.at[hb], semw, add=True)
        mk_cps.append((h1, h2))
    for hs in mk_cps[7:]:
        for h in hs:
            h.wait()
    plsc.subcore_barrier()

    pltpu.sync_copy(
        marker_sp.at[pl.ds(pl.multiple_of(base, 8), CHUNK + 16)], mbuf)
    pltpu.sync_copy(hist_sp, zbuf.at[pl.ds(0, 48)])
    h0 = zbuf[pl.ds(0, 16)]
    h1 = zbuf[pl.ds(16, 16)]
    wsp = i0 + wid
    hsum = jnp.where(lane < wsp, h0, 0) + jnp.where(lane + 16 < wsp, h1, 0)
    for s in (1, 2, 4, 8):
        hsum = hsum + _vtake(hsum, jnp.bitwise_xor(lane, s))
    cnt_base = hsum[0]

    def pv(k, c):
        off = k * 16
        v = mbuf[pl.ds(off, 16)]
        for s in (1, 2, 4, 8):
            sh = _vtake(v, jnp.maximum(lane - s, 0))
            v = jnp.where(lane >= s, v + sh, v)
        v = v + c
        seg_a[pl.ds(off, 16)] = v
        return i0 + v[15]
    lax.fori_loop(0, NV + 1, pv, i0 + (cnt_base - 1))

    m0vec = mbuf[pl.ds(0, 16)]
    crosses = (m0vec[0] == 0).astype(jnp.int32)
    s0vec = seg_a[pl.ds(0, 16)]
    gp = s0vec[0]
    eL = seg_a[pl.ds(CHUNK - 16, 16)]
    gs = eL[15]
    eR = seg_a[pl.ds(CHUNK, 16)]
    crossesR = (eR[0] == gs).astype(jnp.int32)
    own = crossesR * (1 - crosses * (gs == gp).astype(jnp.int32))

    tmp16[pl.ds(0, 16)] = (i0 + gs) + jnp.minimum(lane, 1)
    pltpu.async_copy(nps_h.at[tmp16], widx, semp).wait()
    wv = widx[...]
    s_p = wv[0]
    e_p = wv[1]
    a0 = jnp.bitwise_and(s_p, -8)
    nwin = own * lax.shift_right_logical(e_p - a0 + 15, 4)

    f16 = lambda v: jnp.full((16,), v, jnp.float32)

    def pwin(j, acc):
        axm, axM, aym, ayM = acc
        wbase = pl.multiple_of(a0 + j * 16, 8)
        pltpu.sync_copy(fnpf_h.at[pl.ds(wbase, 16)], widx)
        pltpu.async_copy(pinx_h.at[widx], wx, semp).wait()
        pltpu.async_copy(piny_h.at[widx], wy, semp).wait()
        pp = wbase + lane
        x = wx[...]
        y = wy[...]
        sel = lambda v, fill: jnp.where(
            pp >= s_p, jnp.where(pp < e_p, v, fill), fill)
        return (jnp.minimum(axm, sel(x, BIG)),
                jnp.maximum(axM, sel(x, -BIG)),
                jnp.minimum(aym, sel(y, BIG)),
                jnp.maximum(ayM, sel(y, -BIG)))

    pxm, pxM, pym, pyM = lax.fori_loop(
        0, nwin, pwin, (f16(BIG), f16(-BIG), f16(BIG), f16(-BIG)))
    for s in (1, 2, 4, 8):
        bf = jnp.bitwise_xor(lane, s)
        pxm = jnp.minimum(pxm, _vtake(pxm, bf))
        pxM = jnp.maximum(pxM, _vtake(pxM, bf))
        pym = jnp.minimum(pym, _vtake(pym, bf))
        pyM = jnp.maximum(pyM, _vtake(pyM, bf))
    sm = (own * jnp.maximum(1 - lane, 0)).astype(jnp.float32)
    z16f = jnp.full((16,), 0.0, jnp.float32)
    for buf, val in ((sxm_b, pxm), (sxM_b, pxM), (sym_b, pym), (syM_b, pyM)):
        buf[pl.ds(0, 16)] = val * sm
        buf[pl.ds(16, 16)] = z16f
        buf[pl.ds(32, 16)] = z16f
        buf[pl.ds(48, 16)] = z16f
    so = NUM_PINS + wid * 64
    pcps = [pltpu.async_copy(sxm_b, bxm_h.at[pl.ds(so, 64)], semq),
            pltpu.async_copy(sxM_b, bxM_h.at[pl.ds(so, 64)], semq),
            pltpu.async_copy(sym_b, bym_h.at[pl.ds(so, 64)], semq),
            pltpu.async_copy(syM_b, byM_h.at[pl.ds(so, 64)], semq)]

    for cp in coord_cps:
        cp.wait()

    gpv = i0 + gp
    crvi = i0 + crosses
    carry = (i0 - 1, f16(BIG), f16(-BIG), f16(BIG), f16(-BIG))
    out_cps = []
    for blk in range(16):
        xm_b, xM_b, ym_b, yM_b = bufs[blk % 2]
        if blk >= 2:
            for h in out_cps[blk - 2]:
                h.wait()

        def mainb(kl, c):
            cgv, cxm, cxM, cym, cyM = c
            off = blk * 128 + kl * 16
            x = gx[pl.ds(off, 16)]
            y = gy[pl.ds(off, 16)]
            g = seg_a[pl.ds(off, 16)]
            gnv = seg_a[pl.ds(off + 16, 16)]
            xm, xM, ym, yM = x, x, y, y
            for s in (1, 2, 4, 8):
                idxs = jnp.maximum(lane - s, 0)
                pen = jnp.maximum(jnp.full((16,), s, jnp.int32) - lane, 0)
                ok = (jnp.abs(_vtake(g, idxs) - g) + pen) == 0
                xm = jnp.where(ok, jnp.minimum(xm, _vtake(xm, idxs)), xm)
                xM = jnp.where(ok, jnp.maximum(xM, _vtake(xM, idxs)), xM)
                ym = jnp.where(ok, jnp.minimum(ym, _vtake(ym, idxs)), ym)
                yM = jnp.where(ok, jnp.maximum(yM, _vtake(yM, idxs)), yM)
            mc = g == cgv
            xm = jnp.where(mc, jnp.minimum(xm, cxm), xm)
            xM = jnp.where(mc, jnp.maximum(xM, cxM), xM)
            ym = jnp.where(mc, jnp.minimum(ym, cym), ym)
            yM = jnp.where(mc, jnp.maximum(yM, cyM), yM)
            gnext = jnp.where(lane == 15, i0 + gnv[0],
                              _vtake(g, jnp.minimum(lane + 1, 15)))
            suppress = jnp.where(g == gpv, crvi, i0)
            me_i = jnp.where(gnext != g, 1 - suppress, i0)
            mef = me_i.astype(jnp.float32)
            o = kl * 16
            xm_b[pl.ds(o, 16)] = xm * mef
            xM_b[pl.ds(o, 16)] = xM * mef
            ym_b[pl.ds(o, 16)] = ym * mef
            yM_b[pl.ds(o, 16)] = yM * mef
            return (i0 + g[15], f16(0.0) + xm[15], f16(0.0) + xM[15],
                    f16(0.0) + ym[15], f16(0.0) + yM[15])

        carry = lax.fori_loop(0, 8, mainb, carry)
        bo = base + blk * 128
        out_cps.append([
            pltpu.async_copy(xm_b, bxm_h.at[pl.ds(bo, 128)], semw),
            pltpu.async_copy(xM_b, bxM_h.at[pl.ds(bo, 128)], semw),
            pltpu.async_copy(ym_b, bym_h.at[pl.ds(bo, 128)], semw),
            pltpu.async_copy(yM_b, byM_h.at[pl.ds(bo, 128)], semw)])

    for hs in out_cps[14:]:
        for h in hs:
            h.wait()
    for cp in pcps:
        cp.wait()


def _sc_bbox(pin_x, pin_y, flat_netpin, netpin_start):
    fnp3 = flat_netpin.reshape(W, 16, 128)
    fnpf = jnp.concatenate(
        [flat_netpin, jnp.zeros((FNP_PAD - NUM_PINS,), jnp.int32)])
    nps = jnp.concatenate(
        [netpin_start,
         jnp.full((NPS_PAD - NUM_NETS - 1,), NPS_FILL, jnp.int32)])
    mesh = plsc.VectorSubcoreMesh(core_axis_name="c", subcore_axis_name="s")
    f = pl.kernel(
        _sc_bbox_body,
        out_type=[jax.ShapeDtypeStruct((NREC,), jnp.float32)] * 4,
        mesh=mesh,
        scratch_types=[
            pltpu.VMEM((16, 128), jnp.int32),
            pltpu.VMEM((CHUNK,), jnp.float32),
            pltpu.VMEM((CHUNK,), jnp.float32),
            pltpu.VMEM((CHUNK + 16,), jnp.int32),
            pltpu.VMEM((CHUNK + 16,), jnp.int32),
            pltpu.VMEM((4128,), jnp.int32),
            pltpu.VMEM((128,), jnp.int32),
            pltpu.VMEM((128,), jnp.int32),
            pltpu.VMEM((128,), jnp.int32),
            pltpu.VMEM((128,), jnp.int32),
            pltpu.VMEM((128,), jnp.float32),
            pltpu.VMEM((128,), jnp.float32),
            pltpu.VMEM((128,), jnp.float32),
            pltpu.VMEM((128,), jnp.float32),
            pltpu.VMEM((128,), jnp.float32),
            pltpu.VMEM((128,), jnp.float32),
            pltpu.VMEM((128,), jnp.float32),
            pltpu.VMEM((128,), jnp.float32),
            pltpu.VMEM((16,), jnp.int32),
            pltpu.VMEM((16,), jnp.float32),
            pltpu.VMEM((16,), jnp.float32),
            pltpu.VMEM((64,), jnp.float32),
            pltpu.VMEM((64,), jnp.float32),
            pltpu.VMEM((64,), jnp.float32),
            pltpu.VMEM((64,), jnp.float32),
            pltpu.VMEM((16,), jnp.int32),
            pltpu.VMEM_SHARED((MARKER_N,), jnp.int32),
            pltpu.VMEM_SHARED((48,), jnp.int32),
            pltpu.SemaphoreType.DMA,
            pltpu.SemaphoreType.DMA,
            pltpu.SemaphoreType.DMA,
            pltpu.SemaphoreType.DMA,
            pltpu.SemaphoreType.DMA,
        ],
    )
    return f(pin_x, pin_y, fnp3, fnpf, nps)


def _demand_body(xm_ref, xM_ref, ym_ref, yM_ref, rt_ref, acc_ref):
    i = pl.program_id(0)

    @pl.when(i == 0)
    def _():
        acc_ref[...] = jnp.zeros_like(acc_ref)

    xm = xm_ref[0]
    xM = xM_ref[0]
    ym = ym_ref[0]
    yM = yM_ref[0]
    w = xM - xm
    h = yM - ym
    area = w * h
    pos = area > 0
    safe = jnp.where(pos, area, 1.0)
    dh = jnp.where(pos, w / safe, 0.0)
    dv = jnp.where(pos, h / safe, 0.0)
    b_lo = lax.broadcasted_iota(jnp.int32, (NUM_BINS, TN), 0).astype(
        jnp.float32) * BIN
    ox = jnp.clip(jnp.minimum(xM, b_lo + BIN) - jnp.maximum(xm, b_lo), 0.0, BIN)
    oy = jnp.clip(jnp.minimum(yM, b_lo + BIN) - jnp.maximum(ym, b_lo), 0.0, BIN)
    stacked = jnp.concatenate([ox * dh, ox * dv], axis=0)
    acc_ref[...] += lax.dot_general(
        stacked, oy, (((1,), (1,)), ((), ())),
        preferred_element_type=jnp.float32)

    @pl.when(i == NT_D - 1)
    def _():
        util = acc_ref[...] / (CAP_H * BIN_AREA)
        rt_ref[...] = jnp.clip(
            jnp.maximum(util[:NUM_BINS, :], util[NUM_BINS:, :]),
            MIN_RATE, MAX_RATE)


def _instance_body(rt_ref, nx_ref, ny_ref, sx_ref, sy_ref, out_ref):
    b_lo = lax.broadcasted_iota(jnp.int32, (NUM_BINS, TN), 0).astype(
        jnp.float32) * BIN
    nx = nx_ref[0]
    ny = ny_ref[0]
    nox = jnp.clip(jnp.minimum(nx + sx_ref[0], b_lo + BIN)
                   - jnp.maximum(nx, b_lo), 0.0, BIN)
    noy = jnp.clip(jnp.minimum(ny + sy_ref[0], b_lo + BIN)
                   - jnp.maximum(ny, b_lo), 0.0, BIN)
    t1 = lax.dot_general(rt_ref[...], nox, (((0,), (0,)), ((), ())),
                         preferred_element_type=jnp.float32)
    out_ref[0] = jnp.sum(t1 * noy, axis=0, keepdims=True)


def _dense_pipeline(x_min, x_max, y_min, y_max, nx, ny, sx, sy):
    r2 = lambda a: a.reshape(NT, 1, TN)
    rd = lambda a: a.reshape(NT_D, 1, TN)
    rt = pl.pallas_call(
        _demand_body,
        grid=(NT_D,),
        in_specs=[pl.BlockSpec((1, 1, TN), lambda i: (i, 0, 0))] * 4,
        out_specs=pl.BlockSpec((NUM_BINS, NUM_BINS), lambda i: (0, 0)),
        out_shape=jax.ShapeDtypeStruct((NUM_BINS, NUM_BINS), jnp.float32),
        scratch_shapes=[pltpu.VMEM((2 * NUM_BINS, NUM_BINS), jnp.float32)],
    )(rd(x_min), rd(x_max), rd(y_min), rd(y_max))
    out = pl.pallas_call(
        _instance_body,
        grid=(NT,),
        in_specs=[pl.BlockSpec((NUM_BINS, NUM_BINS), lambda i: (0, 0))]
        + [pl.BlockSpec((1, 1, TN), lambda i: (i, 0, 0))] * 4,
        out_specs=pl.BlockSpec((1, 1, TN), lambda i: (i, 0, 0)),
        out_shape=jax.ShapeDtypeStruct((NT, 1, TN), jnp.float32),
    )(rt, r2(nx), r2(ny), r2(sx), r2(sy))
    return out.reshape(NUM_MOVABLE)


def kernel(pos, pin_pos, node_size_x, node_size_y, netpin_start, flat_netpin):
    pin_x = pin_pos[:NUM_PINS]
    pin_y = pin_pos[NUM_PINS:]
    x_min, x_max, y_min, y_max = _sc_bbox(pin_x, pin_y, flat_netpin,
                                          netpin_start)
    nx = pos[:NUM_MOVABLE]
    ny = pos[NUM_NODES:NUM_NODES + NUM_MOVABLE]
    sx = node_size_x[:NUM_MOVABLE]
    sy = node_size_y[:NUM_MOVABLE]
    return _dense_pipeline(x_min, x_max, y_min, y_max, nx, ny, sx, sy)
